# Initial kernel scaffold; baseline (speedup 1.0000x reference)
import jax
import jax.numpy as jnp
from jax.experimental import pallas as pl


def kernel(x):
    raise NotImplementedError("write your pallas kernel here")


# speedup vs baseline: 1.1546x; 1.1546x over previous
import jax
import jax.numpy as jnp
from jax.experimental import pallas as pl


def _norm_body(x_ref, o_ref):
    xb = x_ref[...]
    s = jnp.sum(xb * xb, axis=1, keepdims=True)
    norm = jnp.sqrt(s)
    o_ref[...] = xb / jnp.maximum(norm, 1e-12)


def kernel(x):
    M, D = x.shape
    BM = 1024
    return pl.pallas_call(
        _norm_body,
        grid=(M // BM,),
        in_specs=[pl.BlockSpec((BM, D), lambda i: (i, 0))],
        out_specs=pl.BlockSpec((BM, D), lambda i: (i, 0)),
        out_shape=jax.ShapeDtypeStruct((M, D), x.dtype),
    )(x)


# speedup vs baseline: 1.1839x; 1.0253x over previous
import jax
import jax.numpy as jnp
from jax.experimental import pallas as pl


def _norm_body(x_ref, o_ref):
    xb = x_ref[...]
    s = jnp.sum(xb * xb, axis=1, keepdims=True)
    r = jax.lax.rsqrt(jnp.maximum(s, 1e-24))
    o_ref[...] = xb * r


def kernel(x):
    M, D = x.shape
    BM = 1024
    return pl.pallas_call(
        _norm_body,
        grid=(M // BM,),
        in_specs=[pl.BlockSpec((BM, D), lambda i: (i, 0))],
        out_specs=pl.BlockSpec((BM, D), lambda i: (i, 0)),
        out_shape=jax.ShapeDtypeStruct((M, D), x.dtype),
    )(x)
